# Initial kernel scaffold; baseline (speedup 1.0000x reference)
import jax
import jax.numpy as jnp
from jax.experimental import pallas as pl


def kernel(X, W1, b1, W2, b2):
    raise NotImplementedError("write your pallas kernel here")


# speedup vs baseline: 7.7953x; 7.7953x over previous
import functools

import jax
import jax.numpy as jnp
from jax import lax
from jax.experimental import pallas as pl
from jax.experimental.pallas import tpu as pltpu
from jax.experimental.pallas import tpu_sc as plsc

N = 4096
OUT = 256
K = 16
BR = 256
NB = N // BR
BE = 128
NBE = N // BE

SC_NC = 2
SC_NS = 16
SC_NW = SC_NC * SC_NS


def _bf16_dot(a, b):
    return lax.dot_general(a.astype(jnp.bfloat16), b.astype(jnp.bfloat16),
                           (((1,), (0,)), ((), ())),
                           preferred_element_type=jnp.float32)


def _topk_body(xbf_ref, xtbf_ref, sq_ref, idx_ref):
    i = pl.program_id(0)
    d = lax.dot_general(xbf_ref[...], xtbf_ref[...], (((1,), (0,)), ((), ())),
                        preferred_element_type=jnp.float32)
    sqr = sq_ref[0, pl.ds(i * BR, BR)][:, None]
    sqc = sq_ref[...]
    d2 = (sqr + sqc) - 2.0 * d
    col = lax.broadcasted_iota(jnp.int32, (BR, N), 1)
    row = lax.broadcasted_iota(jnp.int32, (BR, N), 0) + i * BR
    d2 = jnp.where(col == row, d2 + 1e10, d2)
    cols = []
    for _ in range(K):
        m = jnp.min(d2, axis=1, keepdims=True)
        ismin = d2 == m
        idxk = jnp.min(jnp.where(ismin, col, jnp.int32(1 << 30)), axis=1)
        d2 = jnp.where(col == idxk[:, None], jnp.float32(3e38), d2)
        cols.append(idxk[:, None])
    idx_ref[...] = jnp.concatenate(cols, axis=1)


def _topk(Xbf, XTbf, sq):
    d_in = Xbf.shape[1]
    return pl.pallas_call(
        _topk_body,
        grid=(NB,),
        in_specs=[
            pl.BlockSpec((BR, d_in), lambda i: (i, 0)),
            pl.BlockSpec((d_in, N), lambda i: (0, 0)),
            pl.BlockSpec((1, N), lambda i: (0, 0)),
        ],
        out_specs=pl.BlockSpec((BR, K), lambda i: (i, 0)),
        out_shape=jax.ShapeDtypeStruct((N, K), jnp.int32),
    )(Xbf, XTbf, sq.reshape(1, N))


def _edge_body(x_ref, xj_ref, w1_ref, b1_ref, f_ref):
    x = x_ref[...]
    xbf = x.astype(jnp.bfloat16)
    w1 = w1_ref[...].astype(jnp.bfloat16)
    acc = None
    for k in range(K):
        xjk = xj_ref[k]
        e = jnp.concatenate([xbf, (xjk - x).astype(jnp.bfloat16)], axis=1)
        hk = lax.dot_general(e, w1, (((1,), (0,)), ((), ())),
                             preferred_element_type=jnp.float32)
        acc = hk if acc is None else jnp.maximum(acc, hk)
    f_ref[...] = jnp.maximum(acc + b1_ref[...], 0.0)


def _edgeconv(X, XJ, W1, b1):
    return pl.pallas_call(
        _edge_body,
        grid=(NBE,),
        in_specs=[
            pl.BlockSpec((BE, 128), lambda i: (i, 0)),
            pl.BlockSpec((K, BE, 128), lambda i: (0, i, 0)),
            pl.BlockSpec((2 * 128, OUT), lambda i: (0, 0)),
            pl.BlockSpec((1, OUT), lambda i: (0, 0)),
        ],
        out_specs=pl.BlockSpec((BE, OUT), lambda i: (i, 0)),
        out_shape=jax.ShapeDtypeStruct((N, OUT), jnp.float32),
    )(X, XJ, W1, b1.reshape(1, OUT))


def _final_body(s_ref, w2_ref, b2_ref, o_ref):
    g = _bf16_dot(s_ref[...], w2_ref[...])
    o_ref[...] = jnp.maximum(g + b2_ref[...], 0.0)


def _final(S, W2, b2):
    return pl.pallas_call(
        _final_body,
        grid=(NB,),
        in_specs=[
            pl.BlockSpec((BR, OUT), lambda i: (i, 0)),
            pl.BlockSpec((OUT, OUT), lambda i: (0, 0)),
            pl.BlockSpec((1, OUT), lambda i: (0, 0)),
        ],
        out_specs=pl.BlockSpec((BR, OUT), lambda i: (i, 0)),
        out_shape=jax.ShapeDtypeStruct((N, OUT), jnp.float32),
    )(S, W2, b2.reshape(1, OUT))


def _make_gather(nrows, d, chunk):
    mesh = plsc.VectorSubcoreMesh(core_axis_name="c", subcore_axis_name="s")
    rpw = nrows // SC_NW

    @functools.partial(
        pl.kernel, mesh=mesh,
        out_type=jax.ShapeDtypeStruct((nrows, d), jnp.float32),
        scratch_types=[
            pltpu.VMEM((chunk,), jnp.int32),
            pltpu.VMEM((chunk, d), jnp.float32),
            pltpu.SemaphoreType.DMA,
        ],
    )
    def gather(tab_hbm, idx_hbm, out_hbm, idx_v, rows_v, sem):
        wid = lax.axis_index("s") * SC_NC + lax.axis_index("c")
        base = wid * rpw

        @pl.loop(0, rpw // chunk)
        def _(ch):
            rb = base + ch * chunk
            pltpu.sync_copy(idx_hbm.at[pl.ds(rb, chunk)], idx_v)
            pltpu.async_copy(tab_hbm.at[idx_v], rows_v, sem).wait()
            pltpu.sync_copy(rows_v, out_hbm.at[pl.ds(rb, chunk)])

    return gather


def _make_meanadd(ch_nodes):
    ncols = OUT // 16
    mesh = plsc.VectorSubcoreMesh(core_axis_name="c", subcore_axis_name="s")
    npw = N // SC_NW

    @functools.partial(
        pl.kernel, mesh=mesh,
        out_type=jax.ShapeDtypeStruct((N, OUT), jnp.float32),
        scratch_types=[
            pltpu.VMEM((ch_nodes * K,), jnp.int32),
            pltpu.VMEM((ch_nodes * K, OUT), jnp.float32),
            pltpu.VMEM((ch_nodes, OUT), jnp.float32),
            pltpu.VMEM((ch_nodes, OUT), jnp.float32),
            pltpu.SemaphoreType.DMA,
        ],
    )
    def meanadd(a_hbm, b_hbm, idx_hbm, out_hbm, idx_v, rows_v, a_v, o_v, sem):
        wid = lax.axis_index("s") * SC_NC + lax.axis_index("c")
        base = wid * npw

        @pl.loop(0, npw // ch_nodes)
        def _(ch):
            nb = base + ch * ch_nodes
            pltpu.sync_copy(idx_hbm.at[pl.ds(nb * K, ch_nodes * K)], idx_v)
            pltpu.async_copy(b_hbm.at[idx_v], rows_v, sem).wait()
            pltpu.sync_copy(a_hbm.at[pl.ds(nb, ch_nodes)], a_v)
            for n in range(ch_nodes):
                for c in range(ncols):
                    sl = pl.ds(c * 16, 16)
                    acc = rows_v[n * K, sl]
                    for j in range(1, K):
                        acc = acc + rows_v[n * K + j, sl]
                    o_v[n, sl] = acc * jnp.float32(1.0 / K) + a_v[n, sl]
            pltpu.sync_copy(o_v, out_hbm.at[pl.ds(nb, ch_nodes)])

    return meanadd


_make_gather = functools.lru_cache(maxsize=None)(_make_gather)
_make_meanadd = functools.lru_cache(maxsize=None)(_make_meanadd)


def kernel(X, W1, b1, W2, b2):
    Xbf = X.astype(jnp.bfloat16)
    sq1 = jnp.sum(X * X, axis=-1)
    idx1 = _topk(Xbf, X.T.astype(jnp.bfloat16), sq1)
    idx1_km = idx1.T.reshape(-1)
    XJ = _make_gather(N * K, 128, 256)(X, idx1_km).reshape(K, N, 128)
    F = _edgeconv(X, XJ, W1, b1)
    sq2 = jnp.sum(F * F, axis=-1)
    idx2 = _topk(F.astype(jnp.bfloat16), F.T.astype(jnp.bfloat16), sq2)
    S = _make_meanadd(8)(F, F, idx2.reshape(-1))
    return _final(S, W2, b2)


# speedup vs baseline: 9.7985x; 1.2570x over previous
import functools

import jax
import jax.numpy as jnp
from jax import lax
from jax.experimental import pallas as pl
from jax.experimental.pallas import tpu as pltpu
from jax.experimental.pallas import tpu_sc as plsc

N = 4096
OUT = 256
K = 16
BR = 256
NB = N // BR
BE = 128
NBE = N // BE

SC_NC = 2
SC_NS = 16
SC_NW = SC_NC * SC_NS


def _bf16_dot(a, b):
    return lax.dot_general(a.astype(jnp.bfloat16), b.astype(jnp.bfloat16),
                           (((1,), (0,)), ((), ())),
                           preferred_element_type=jnp.float32)


def _topk_body(xbf_ref, xtbf_ref, sq_ref, idx_ref):
    i = pl.program_id(0)
    d = lax.dot_general(xbf_ref[...], xtbf_ref[...], (((1,), (0,)), ((), ())),
                        preferred_element_type=jnp.float32)
    sqr = sq_ref[0, pl.ds(i * BR, BR)][:, None]
    sqc = sq_ref[...]
    d2 = (sqr + sqc) - 2.0 * d
    colf = lax.broadcasted_iota(jnp.int32, (BR, N), 1).astype(jnp.float32)
    rowf = (lax.broadcasted_iota(jnp.int32, (BR, N), 0) + i * BR).astype(jnp.float32)
    d2 = jnp.where(colf == rowf, d2 + 1e10, d2)
    big = jnp.float32(3e38)
    cols = []
    for _ in range(K):
        m = jnp.min(d2, axis=1, keepdims=True)
        ismin = d2 == m
        idxk = jnp.min(jnp.where(ismin, colf, big), axis=1)
        d2 = jnp.where(colf == idxk[:, None], big, d2)
        cols.append(idxk[:, None])
    idx_ref[...] = jnp.concatenate(cols, axis=1).astype(jnp.int32)


def _topk(Xbf, XTbf, sq):
    d_in = Xbf.shape[1]
    return pl.pallas_call(
        _topk_body,
        grid=(NB,),
        in_specs=[
            pl.BlockSpec((BR, d_in), lambda i: (i, 0)),
            pl.BlockSpec((d_in, N), lambda i: (0, 0)),
            pl.BlockSpec((1, N), lambda i: (0, 0)),
        ],
        out_specs=pl.BlockSpec((BR, K), lambda i: (i, 0)),
        out_shape=jax.ShapeDtypeStruct((N, K), jnp.int32),
    )(Xbf, XTbf, sq.reshape(1, N))


def _edge_body(x_ref, xj_ref, w1_ref, b1_ref, f_ref):
    x = x_ref[...]
    xbf = x.astype(jnp.bfloat16)
    w1 = w1_ref[...].astype(jnp.bfloat16)
    acc = None
    for k in range(K):
        xjk = xj_ref[k]
        e = jnp.concatenate([xbf, (xjk - x).astype(jnp.bfloat16)], axis=1)
        hk = lax.dot_general(e, w1, (((1,), (0,)), ((), ())),
                             preferred_element_type=jnp.float32)
        acc = hk if acc is None else jnp.maximum(acc, hk)
    f_ref[...] = jnp.maximum(acc + b1_ref[...], 0.0)


def _edgeconv(X, XJ, W1, b1):
    return pl.pallas_call(
        _edge_body,
        grid=(NBE,),
        in_specs=[
            pl.BlockSpec((BE, 128), lambda i: (i, 0)),
            pl.BlockSpec((K, BE, 128), lambda i: (0, i, 0)),
            pl.BlockSpec((2 * 128, OUT), lambda i: (0, 0)),
            pl.BlockSpec((1, OUT), lambda i: (0, 0)),
        ],
        out_specs=pl.BlockSpec((BE, OUT), lambda i: (i, 0)),
        out_shape=jax.ShapeDtypeStruct((N, OUT), jnp.float32),
    )(X, XJ, W1, b1.reshape(1, OUT))


def _final_body(s_ref, w2_ref, b2_ref, o_ref):
    g = _bf16_dot(s_ref[...], w2_ref[...])
    o_ref[...] = jnp.maximum(g + b2_ref[...], 0.0)


def _final(S, W2, b2):
    return pl.pallas_call(
        _final_body,
        grid=(NB,),
        in_specs=[
            pl.BlockSpec((BR, OUT), lambda i: (i, 0)),
            pl.BlockSpec((OUT, OUT), lambda i: (0, 0)),
            pl.BlockSpec((1, OUT), lambda i: (0, 0)),
        ],
        out_specs=pl.BlockSpec((BR, OUT), lambda i: (i, 0)),
        out_shape=jax.ShapeDtypeStruct((N, OUT), jnp.float32),
    )(S, W2, b2.reshape(1, OUT))


def _make_gather(nrows, d, chunk, tab_rows):
    mesh = plsc.VectorSubcoreMesh(core_axis_name="c", subcore_axis_name="s")
    rpw = nrows // SC_NW
    nch = rpw // chunk
    assert nch % 2 == 0

    @functools.partial(
        pl.kernel, mesh=mesh,
        out_type=jax.ShapeDtypeStruct((nrows, d), jnp.float32),
        scratch_types=[
            pltpu.VMEM((chunk,), jnp.int32),
            pltpu.VMEM((chunk,), jnp.int32),
            pltpu.VMEM((chunk, d), jnp.float32),
            pltpu.VMEM((chunk, d), jnp.float32),
            pltpu.VMEM_SHARED((tab_rows, d), jnp.float32),
            pltpu.SemaphoreType.DMA,
            pltpu.SemaphoreType.DMA,
            pltpu.SemaphoreType.DMA,
        ],
    )
    def gather(tab_hbm, idx_hbm, out_hbm, idx_v0, idx_v1, rows_v0, rows_v1,
               shared, gsem0, gsem1, osem):
        sid = lax.axis_index("s")
        wid = sid * SC_NC + lax.axis_index("c")
        base = wid * rpw
        idx_v = (idx_v0, idx_v1)
        rows_v = (rows_v0, rows_v1)
        gsems = (gsem0, gsem1)

        @pl.when(sid == 0)
        def _():
            pltpu.sync_copy(tab_hbm, shared)

        plsc.subcore_barrier()

        def idx_load(c, b):
            pltpu.sync_copy(idx_hbm.at[pl.ds(base + c * chunk, chunk)],
                            idx_v[b])

        def g_start(b):
            pltpu.async_copy(shared.at[idx_v[b]], rows_v[b], gsems[b])

        def g_wait(b):
            pltpu.make_async_copy(shared.at[idx_v[b]], rows_v[b],
                                  gsems[b]).wait()

        def o_drain(b):
            pltpu.make_async_copy(rows_v[b],
                                  out_hbm.at[pl.ds(base, chunk)], osem).wait()

        idx_load(0, 0)
        g_start(0)

        @pl.loop(0, nch // 2)
        def _(h):
            for b in (0, 1):
                c = h * 2 + b
                cn = jnp.minimum(c + 1, nch - 1)
                idx_load(cn, 1 - b)

                @pl.when(c >= 1)
                def _():
                    o_drain(b)

                g_start(1 - b)
                g_wait(b)
                pltpu.async_copy(rows_v[b],
                                 out_hbm.at[pl.ds(base + c * chunk, chunk)],
                                 osem)

        o_drain(0)
        g_wait(0)

    return gather


def _make_meanadd(ch_nodes):
    ncols = OUT // 16
    mesh = plsc.VectorSubcoreMesh(core_axis_name="c", subcore_axis_name="s")
    npw = N // SC_NW
    nch = npw // ch_nodes
    assert nch % 2 == 0
    chk = ch_nodes * K

    @functools.partial(
        pl.kernel, mesh=mesh,
        out_type=jax.ShapeDtypeStruct((N, OUT), jnp.float32),
        scratch_types=[
            pltpu.VMEM((chk,), jnp.int32),
            pltpu.VMEM((chk,), jnp.int32),
            pltpu.VMEM((chk, OUT), jnp.float32),
            pltpu.VMEM((chk, OUT), jnp.float32),
            pltpu.VMEM((ch_nodes, OUT), jnp.float32),
            pltpu.VMEM((ch_nodes, OUT), jnp.float32),
            pltpu.SemaphoreType.DMA,
            pltpu.SemaphoreType.DMA,
        ],
    )
    def meanadd(a_hbm, b_hbm, idx_hbm, out_hbm,
                idx_v0, idx_v1, rows_v0, rows_v1, a_v, o_v,
                gsem0, gsem1):
        sid = lax.axis_index("s")
        wid = sid * SC_NC + lax.axis_index("c")
        base = wid * npw
        idx_v = (idx_v0, idx_v1)
        rows_v = (rows_v0, rows_v1)
        gsems = (gsem0, gsem1)

        def idx_load(c, b):
            pltpu.sync_copy(idx_hbm.at[pl.ds((base + c * ch_nodes) * K, chk)],
                            idx_v[b])

        def g_start(b):
            pltpu.async_copy(b_hbm.at[idx_v[b]], rows_v[b], gsems[b])

        def g_wait(b):
            pltpu.make_async_copy(b_hbm.at[idx_v[b]], rows_v[b],
                                  gsems[b]).wait()

        idx_load(0, 0)
        g_start(0)

        @pl.loop(0, nch // 2)
        def _(h):
            for b in (0, 1):
                c = h * 2 + b
                cn = jnp.minimum(c + 1, nch - 1)
                nb = base + c * ch_nodes
                idx_load(cn, 1 - b)
                g_start(1 - b)
                pltpu.sync_copy(a_hbm.at[pl.ds(nb, ch_nodes)], a_v)
                g_wait(b)
                rv = rows_v[b]
                for n in range(ch_nodes):
                    for cc in range(ncols):
                        sl = pl.ds(cc * 16, 16)
                        acc = rv[n * K, sl]
                        for j in range(1, K):
                            acc = acc + rv[n * K + j, sl]
                        o_v[n, sl] = acc * jnp.float32(1.0 / K) + a_v[n, sl]
                pltpu.sync_copy(o_v, out_hbm.at[pl.ds(nb, ch_nodes)])

        g_wait(0)

    return meanadd


_make_gather = functools.lru_cache(maxsize=None)(_make_gather)
_make_meanadd = functools.lru_cache(maxsize=None)(_make_meanadd)


def kernel(X, W1, b1, W2, b2):
    Xbf = X.astype(jnp.bfloat16)
    sq1 = jnp.sum(X * X, axis=-1)
    idx1 = _topk(Xbf, X.T.astype(jnp.bfloat16), sq1)
    idx1_km = idx1.T.reshape(-1)
    XJ = _make_gather(N * K, 128, 256, N)(X, idx1_km).reshape(K, N, 128)
    F = _edgeconv(X, XJ, W1, b1)
    sq2 = jnp.sum(F * F, axis=-1)
    idx2 = _topk(F.astype(jnp.bfloat16), F.T.astype(jnp.bfloat16), sq2)
    S = _make_meanadd(8)(F, F, idx2.reshape(-1))
    return _final(S, W2, b2)


# speedup vs baseline: 10.2241x; 1.0434x over previous
import functools

import jax
import jax.numpy as jnp
from jax import lax
from jax.experimental import pallas as pl
from jax.experimental.pallas import tpu as pltpu
from jax.experimental.pallas import tpu_sc as plsc

N = 4096
OUT = 256
K = 16
BR = 256
NB = N // BR
BE = 128
NBE = N // BE

SC_NC = 2
SC_NS = 16
SC_NW = SC_NC * SC_NS


def _bf16_dot(a, b):
    return lax.dot_general(a.astype(jnp.bfloat16), b.astype(jnp.bfloat16),
                           (((1,), (0,)), ((), ())),
                           preferred_element_type=jnp.float32)


def _topk_body(xbf_ref, xtbf_ref, sq_ref, idx_ref):
    i = pl.program_id(0)
    d = lax.dot_general(xbf_ref[...], xtbf_ref[...], (((1,), (0,)), ((), ())),
                        preferred_element_type=jnp.float32)
    sqr = sq_ref[0, pl.ds(i * BR, BR)][:, None]
    sqc = sq_ref[...]
    d2 = (sqr + sqc) - 2.0 * d
    colf = lax.broadcasted_iota(jnp.int32, (BR, N), 1).astype(jnp.float32)
    rowf = (lax.broadcasted_iota(jnp.int32, (BR, N), 0) + i * BR).astype(jnp.float32)
    d2 = jnp.where(colf == rowf, d2 + 1e10, d2)
    big = jnp.float32(3e38)
    cols = []
    for _ in range(K):
        m = jnp.min(d2, axis=1, keepdims=True)
        ismin = d2 == m
        idxk = jnp.min(jnp.where(ismin, colf, big), axis=1)
        d2 = jnp.where(colf == idxk[:, None], big, d2)
        cols.append(idxk[:, None])
    idx_ref[...] = jnp.concatenate(cols, axis=1).astype(jnp.int32)


def _topk(Xbf, XTbf, sq):
    d_in = Xbf.shape[1]
    return pl.pallas_call(
        _topk_body,
        grid=(NB,),
        in_specs=[
            pl.BlockSpec((BR, d_in), lambda i: (i, 0)),
            pl.BlockSpec((d_in, N), lambda i: (0, 0)),
            pl.BlockSpec((1, N), lambda i: (0, 0)),
        ],
        out_specs=pl.BlockSpec((BR, K), lambda i: (i, 0)),
        out_shape=jax.ShapeDtypeStruct((N, K), jnp.int32),
    )(Xbf, XTbf, sq.reshape(1, N))


def _edge_body(x_ref, xj_ref, w1_ref, b1_ref, f_ref):
    x = x_ref[...]
    xbf = x.astype(jnp.bfloat16)
    w1 = w1_ref[...].astype(jnp.bfloat16)
    acc = None
    for k in range(K):
        xjk = xj_ref[k]
        e = jnp.concatenate([xbf, (xjk - x).astype(jnp.bfloat16)], axis=1)
        hk = lax.dot_general(e, w1, (((1,), (0,)), ((), ())),
                             preferred_element_type=jnp.float32)
        acc = hk if acc is None else jnp.maximum(acc, hk)
    f_ref[...] = jnp.maximum(acc + b1_ref[...], 0.0)


def _edgeconv(X, XJ, W1, b1):
    return pl.pallas_call(
        _edge_body,
        grid=(NBE,),
        in_specs=[
            pl.BlockSpec((BE, 128), lambda i: (i, 0)),
            pl.BlockSpec((K, BE, 128), lambda i: (0, i, 0)),
            pl.BlockSpec((2 * 128, OUT), lambda i: (0, 0)),
            pl.BlockSpec((1, OUT), lambda i: (0, 0)),
        ],
        out_specs=pl.BlockSpec((BE, OUT), lambda i: (i, 0)),
        out_shape=jax.ShapeDtypeStruct((N, OUT), jnp.float32),
    )(X, XJ, W1, b1.reshape(1, OUT))


def _final_body(s_ref, w2_ref, b2_ref, o_ref):
    g = _bf16_dot(s_ref[...], w2_ref[...])
    o_ref[...] = jnp.maximum(g + b2_ref[...], 0.0)


def _final(S, W2, b2):
    return pl.pallas_call(
        _final_body,
        grid=(NB,),
        in_specs=[
            pl.BlockSpec((BR, OUT), lambda i: (i, 0)),
            pl.BlockSpec((OUT, OUT), lambda i: (0, 0)),
            pl.BlockSpec((1, OUT), lambda i: (0, 0)),
        ],
        out_specs=pl.BlockSpec((BR, OUT), lambda i: (i, 0)),
        out_shape=jax.ShapeDtypeStruct((N, OUT), jnp.float32),
    )(S, W2, b2.reshape(1, OUT))


def _make_gather(nrows, d, chunk, tab_rows):
    mesh = plsc.VectorSubcoreMesh(core_axis_name="c", subcore_axis_name="s")
    rpw = nrows // SC_NW
    nch = rpw // chunk
    assert nch % 2 == 0

    @functools.partial(
        pl.kernel, mesh=mesh,
        out_type=jax.ShapeDtypeStruct((nrows, d), jnp.float32),
        scratch_types=[
            pltpu.VMEM((chunk,), jnp.int32),
            pltpu.VMEM((chunk,), jnp.int32),
            pltpu.VMEM((chunk, d), jnp.float32),
            pltpu.VMEM((chunk, d), jnp.float32),
            pltpu.VMEM_SHARED((tab_rows, d), jnp.float32),
            pltpu.SemaphoreType.DMA,
            pltpu.SemaphoreType.DMA,
            pltpu.SemaphoreType.DMA,
        ],
    )
    def gather(tab_hbm, idx_hbm, out_hbm, idx_v0, idx_v1, rows_v0, rows_v1,
               shared, gsem0, gsem1, osem):
        sid = lax.axis_index("s")
        wid = sid * SC_NC + lax.axis_index("c")
        base = wid * rpw
        idx_v = (idx_v0, idx_v1)
        rows_v = (rows_v0, rows_v1)
        gsems = (gsem0, gsem1)

        @pl.when(sid == 0)
        def _():
            pltpu.sync_copy(tab_hbm, shared)

        plsc.subcore_barrier()

        def idx_load(c, b):
            pltpu.sync_copy(idx_hbm.at[pl.ds(base + c * chunk, chunk)],
                            idx_v[b])

        def g_start(b):
            pltpu.async_copy(shared.at[idx_v[b]], rows_v[b], gsems[b])

        def g_wait(b):
            pltpu.make_async_copy(shared.at[idx_v[b]], rows_v[b],
                                  gsems[b]).wait()

        def o_drain(b):
            pltpu.make_async_copy(rows_v[b],
                                  out_hbm.at[pl.ds(base, chunk)], osem).wait()

        idx_load(0, 0)
        g_start(0)

        @pl.loop(0, nch // 2)
        def _(h):
            for b in (0, 1):
                c = h * 2 + b
                cn = jnp.minimum(c + 1, nch - 1)
                idx_load(cn, 1 - b)

                @pl.when(c >= 1)
                def _():
                    o_drain(b)

                g_start(1 - b)
                g_wait(b)
                pltpu.async_copy(rows_v[b],
                                 out_hbm.at[pl.ds(base + c * chunk, chunk)],
                                 osem)

        o_drain(0)
        g_wait(0)

    return gather


def _make_meanadd(ch_nodes):
    HD = OUT // SC_NC
    ncols = HD // 16
    mesh = plsc.VectorSubcoreMesh(core_axis_name="c", subcore_axis_name="s")
    npw = N // SC_NS
    nch = npw // ch_nodes
    assert nch % 2 == 0
    chk = ch_nodes * K

    @functools.partial(
        pl.kernel, mesh=mesh,
        out_type=jax.ShapeDtypeStruct((N, OUT), jnp.float32),
        scratch_types=[
            pltpu.VMEM((chk,), jnp.int32),
            pltpu.VMEM((chk,), jnp.int32),
            pltpu.VMEM((chk, HD), jnp.float32),
            pltpu.VMEM((chk, HD), jnp.float32),
            pltpu.VMEM((ch_nodes, HD), jnp.float32),
            pltpu.VMEM((ch_nodes, HD), jnp.float32),
            pltpu.VMEM_SHARED((N, HD), jnp.float32),
            pltpu.SemaphoreType.DMA,
            pltpu.SemaphoreType.DMA,
        ],
    )
    def meanadd(a_hbm, b_hbm, idx_hbm, out_hbm,
                idx_v0, idx_v1, rows_v0, rows_v1, a_v, o_v, shared,
                gsem0, gsem1):
        sid = lax.axis_index("s")
        cid = lax.axis_index("c")
        coff = cid * HD
        base = sid * npw
        idx_v = (idx_v0, idx_v1)
        rows_v = (rows_v0, rows_v1)
        gsems = (gsem0, gsem1)

        @pl.when(sid == 0)
        def _():
            pltpu.sync_copy(b_hbm.at[pl.ds(0, N), pl.ds(coff, HD)], shared)

        plsc.subcore_barrier()

        def idx_load(c, b):
            pltpu.sync_copy(idx_hbm.at[pl.ds((base + c * ch_nodes) * K, chk)],
                            idx_v[b])

        def g_start(b):
            pltpu.async_copy(shared.at[idx_v[b]], rows_v[b], gsems[b])

        def g_wait(b):
            pltpu.make_async_copy(shared.at[idx_v[b]], rows_v[b],
                                  gsems[b]).wait()

        idx_load(0, 0)
        g_start(0)

        @pl.loop(0, nch // 2)
        def _(h):
            for b in (0, 1):
                c = h * 2 + b
                cn = jnp.minimum(c + 1, nch - 1)
                nb = base + c * ch_nodes
                idx_load(cn, 1 - b)
                g_start(1 - b)
                pltpu.sync_copy(
                    a_hbm.at[pl.ds(nb, ch_nodes), pl.ds(coff, HD)], a_v)
                g_wait(b)
                rv = rows_v[b]
                for n in range(ch_nodes):
                    for cc in range(ncols):
                        sl = pl.ds(cc * 16, 16)
                        acc = rv[n * K, sl]
                        for j in range(1, K):
                            acc = acc + rv[n * K + j, sl]
                        o_v[n, sl] = acc * jnp.float32(1.0 / K) + a_v[n, sl]
                pltpu.sync_copy(
                    o_v, out_hbm.at[pl.ds(nb, ch_nodes), pl.ds(coff, HD)])

        g_wait(0)

    return meanadd


_make_gather = functools.lru_cache(maxsize=None)(_make_gather)
_make_meanadd = functools.lru_cache(maxsize=None)(_make_meanadd)


def kernel(X, W1, b1, W2, b2):
    Xbf = X.astype(jnp.bfloat16)
    sq1 = jnp.sum(X * X, axis=-1)
    idx1 = _topk(Xbf, X.T.astype(jnp.bfloat16), sq1)
    idx1_km = idx1.T.reshape(-1)
    XJ = _make_gather(N * K, 128, 256, N)(X, idx1_km).reshape(K, N, 128)
    F = _edgeconv(X, XJ, W1, b1)
    sq2 = jnp.sum(F * F, axis=-1)
    idx2 = _topk(F.astype(jnp.bfloat16), F.T.astype(jnp.bfloat16), sq2)
    S = _make_meanadd(8)(F, F, idx2.reshape(-1))
    return _final(S, W2, b2)


# speedup vs baseline: 10.5509x; 1.0320x over previous
import functools

import jax
import jax.numpy as jnp
from jax import lax
from jax.experimental import pallas as pl
from jax.experimental.pallas import tpu as pltpu
from jax.experimental.pallas import tpu_sc as plsc

N = 4096
OUT = 256
K = 16
BR = 256
NB = N // BR
BE = 128
NBE = N // BE

SC_NC = 2
SC_NS = 16
SC_NW = SC_NC * SC_NS


def _bf16_dot(a, b):
    return lax.dot_general(a.astype(jnp.bfloat16), b.astype(jnp.bfloat16),
                           (((1,), (0,)), ((), ())),
                           preferred_element_type=jnp.float32)


def _topk_body(xbf_ref, xtbf_ref, sq_ref, idx_ref):
    i = pl.program_id(0)
    d = lax.dot_general(xbf_ref[...], xtbf_ref[...], (((1,), (0,)), ((), ())),
                        preferred_element_type=jnp.float32)
    sqr = sq_ref[0, pl.ds(i * BR, BR)][:, None]
    sqc = sq_ref[...]
    d2 = (sqr + sqc) - 2.0 * d
    col = lax.broadcasted_iota(jnp.int32, (BR, N), 1)
    row = lax.broadcasted_iota(jnp.int32, (BR, N), 0) + i * BR
    d2 = jnp.where(col == row, d2 + 1e10, d2)
    H = N // 2
    va = lax.slice(d2, (0, 0), (BR, H))
    vb = lax.slice(d2, (0, H), (BR, N))
    colh = lax.broadcasted_iota(jnp.int32, (BR, H), 1).astype(jnp.float32)
    swap = vb < va
    pm = jnp.where(swap, vb, va)
    pm2 = jnp.where(swap, va, vb)
    oi = jnp.where(swap, colh + H, colh)
    oi2 = jnp.where(swap, colh, colh + H)
    big = jnp.float32(3e38)
    cols = []
    for _ in range(K):
        m = jnp.min(pm, axis=1, keepdims=True)
        hit = pm == m
        idxk = jnp.min(jnp.where(hit, oi, big), axis=1)
        mask = oi == idxk[:, None]
        pm = jnp.where(mask, pm2, pm)
        oi = jnp.where(mask, oi2, oi)
        pm2 = jnp.where(mask, big, pm2)
        cols.append(idxk[:, None])
    idx_ref[...] = jnp.concatenate(cols, axis=1).astype(jnp.int32)


def _topk(Xbf, XTbf, sq):
    d_in = Xbf.shape[1]
    return pl.pallas_call(
        _topk_body,
        grid=(NB,),
        in_specs=[
            pl.BlockSpec((BR, d_in), lambda i: (i, 0)),
            pl.BlockSpec((d_in, N), lambda i: (0, 0)),
            pl.BlockSpec((1, N), lambda i: (0, 0)),
        ],
        out_specs=pl.BlockSpec((BR, K), lambda i: (i, 0)),
        out_shape=jax.ShapeDtypeStruct((N, K), jnp.int32),
    )(Xbf, XTbf, sq.reshape(1, N))


def _edge_body(x_ref, xj_ref, w1_ref, b1_ref, f_ref):
    x = x_ref[...]
    xbf = x.astype(jnp.bfloat16)
    w1 = w1_ref[...].astype(jnp.bfloat16)
    acc = None
    for k in range(K):
        xjk = xj_ref[k]
        e = jnp.concatenate([xbf, (xjk - x).astype(jnp.bfloat16)], axis=1)
        hk = lax.dot_general(e, w1, (((1,), (0,)), ((), ())),
                             preferred_element_type=jnp.float32)
        acc = hk if acc is None else jnp.maximum(acc, hk)
    f_ref[...] = jnp.maximum(acc + b1_ref[...], 0.0)


def _edgeconv(X, XJ, W1, b1):
    return pl.pallas_call(
        _edge_body,
        grid=(NBE,),
        in_specs=[
            pl.BlockSpec((BE, 128), lambda i: (i, 0)),
            pl.BlockSpec((K, BE, 128), lambda i: (0, i, 0)),
            pl.BlockSpec((2 * 128, OUT), lambda i: (0, 0)),
            pl.BlockSpec((1, OUT), lambda i: (0, 0)),
        ],
        out_specs=pl.BlockSpec((BE, OUT), lambda i: (i, 0)),
        out_shape=jax.ShapeDtypeStruct((N, OUT), jnp.float32),
    )(X, XJ, W1, b1.reshape(1, OUT))


def _final_body(s_ref, w2_ref, b2_ref, o_ref):
    g = _bf16_dot(s_ref[...], w2_ref[...])
    o_ref[...] = jnp.maximum(g + b2_ref[...], 0.0)


def _final(S, W2, b2):
    return pl.pallas_call(
        _final_body,
        grid=(NB,),
        in_specs=[
            pl.BlockSpec((BR, OUT), lambda i: (i, 0)),
            pl.BlockSpec((OUT, OUT), lambda i: (0, 0)),
            pl.BlockSpec((1, OUT), lambda i: (0, 0)),
        ],
        out_specs=pl.BlockSpec((BR, OUT), lambda i: (i, 0)),
        out_shape=jax.ShapeDtypeStruct((N, OUT), jnp.float32),
    )(S, W2, b2.reshape(1, OUT))


def _make_gather(nrows, d, chunk, tab_rows):
    mesh = plsc.VectorSubcoreMesh(core_axis_name="c", subcore_axis_name="s")
    rpw = nrows // SC_NW
    nch = rpw // chunk
    assert nch % 2 == 0

    @functools.partial(
        pl.kernel, mesh=mesh,
        out_type=jax.ShapeDtypeStruct((nrows, d), jnp.float32),
        scratch_types=[
            pltpu.VMEM((chunk,), jnp.int32),
            pltpu.VMEM((chunk,), jnp.int32),
            pltpu.VMEM((chunk, d), jnp.float32),
            pltpu.VMEM((chunk, d), jnp.float32),
            pltpu.VMEM_SHARED((tab_rows, d), jnp.float32),
            pltpu.SemaphoreType.DMA,
            pltpu.SemaphoreType.DMA,
            pltpu.SemaphoreType.DMA,
        ],
    )
    def gather(tab_hbm, idx_hbm, out_hbm, idx_v0, idx_v1, rows_v0, rows_v1,
               shared, gsem0, gsem1, osem):
        sid = lax.axis_index("s")
        wid = sid * SC_NC + lax.axis_index("c")
        base = wid * rpw
        idx_v = (idx_v0, idx_v1)
        rows_v = (rows_v0, rows_v1)
        gsems = (gsem0, gsem1)

        @pl.when(sid == 0)
        def _():
            pltpu.sync_copy(tab_hbm, shared)

        plsc.subcore_barrier()

        def idx_load(c, b):
            pltpu.sync_copy(idx_hbm.at[pl.ds(base + c * chunk, chunk)],
                            idx_v[b])

        def g_start(b):
            pltpu.async_copy(shared.at[idx_v[b]], rows_v[b], gsems[b])

        def g_wait(b):
            pltpu.make_async_copy(shared.at[idx_v[b]], rows_v[b],
                                  gsems[b]).wait()

        def o_drain(b):
            pltpu.make_async_copy(rows_v[b],
                                  out_hbm.at[pl.ds(base, chunk)], osem).wait()

        idx_load(0, 0)
        g_start(0)

        @pl.loop(0, nch // 2)
        def _(h):
            for b in (0, 1):
                c = h * 2 + b
                cn = jnp.minimum(c + 1, nch - 1)
                idx_load(cn, 1 - b)

                @pl.when(c >= 1)
                def _():
                    o_drain(b)

                g_start(1 - b)
                g_wait(b)
                pltpu.async_copy(rows_v[b],
                                 out_hbm.at[pl.ds(base + c * chunk, chunk)],
                                 osem)

        o_drain(0)
        g_wait(0)

    return gather


def _make_meanadd(ch_nodes):
    HD = OUT // SC_NC
    ncols = HD // 16
    mesh = plsc.VectorSubcoreMesh(core_axis_name="c", subcore_axis_name="s")
    npw = N // SC_NS
    nch = npw // ch_nodes
    assert nch % 2 == 0
    chk = ch_nodes * K

    assert nch % 4 == 0
    scratch = ([pltpu.VMEM((chk,), jnp.int32)] * 4 +
               [pltpu.VMEM((chk, HD), jnp.float32)] * 2 +
               [pltpu.VMEM((ch_nodes, HD), jnp.float32)] * 4 +
               [pltpu.VMEM_SHARED((N, HD), jnp.float32)] +
               [pltpu.SemaphoreType.DMA] * 9)

    @functools.partial(
        pl.kernel, mesh=mesh,
        out_type=jax.ShapeDtypeStruct((N, OUT), jnp.float32),
        scratch_types=scratch,
    )
    def meanadd(a_hbm, b_hbm, idx_hbm, out_hbm,
                ix0, ix1, ix2, ix3, rows_v0, rows_v1, a_v0, a_v1,
                o_v0, o_v1, shared,
                is0, is1, is2, is3, gsem0, gsem1, asem0, asem1, osem):
        sid = lax.axis_index("s")
        cid = lax.axis_index("c")
        coff = cid * HD
        base = sid * npw
        idx_v = (ix0, ix1, ix2, ix3)
        isems = (is0, is1, is2, is3)
        rows_v = (rows_v0, rows_v1)
        a_v = (a_v0, a_v1)
        o_v = (o_v0, o_v1)
        gsems = (gsem0, gsem1)
        asems = (asem0, asem1)

        @pl.when(sid == 0)
        def _():
            pltpu.sync_copy(b_hbm.at[pl.ds(0, N), pl.ds(coff, HD)], shared)

        plsc.subcore_barrier()

        def idx_start(c, i4):
            pltpu.async_copy(
                idx_hbm.at[pl.ds((base + c * ch_nodes) * K, chk)],
                idx_v[i4], isems[i4])

        def idx_wait(c, i4):
            pltpu.make_async_copy(
                idx_hbm.at[pl.ds((base + c * ch_nodes) * K, chk)],
                idx_v[i4], isems[i4]).wait()

        def g_start(i4, b):
            pltpu.async_copy(shared.at[idx_v[i4]], rows_v[b], gsems[b])

        def g_wait(i4, b):
            pltpu.make_async_copy(shared.at[idx_v[i4]], rows_v[b],
                                  gsems[b]).wait()

        def a_start(c, b):
            pltpu.async_copy(
                a_hbm.at[pl.ds(base + c * ch_nodes, ch_nodes),
                         pl.ds(coff, HD)], a_v[b], asems[b])

        def a_wait(c, b):
            pltpu.make_async_copy(
                a_hbm.at[pl.ds(base + c * ch_nodes, ch_nodes),
                         pl.ds(coff, HD)], a_v[b], asems[b]).wait()

        def o_slice(c):
            return out_hbm.at[pl.ds(base + c * ch_nodes, ch_nodes),
                              pl.ds(coff, HD)]

        idx_start(0, 0)
        idx_start(1, 1)
        idx_wait(0, 0)
        g_start(0, 0)
        a_start(0, 0)

        @pl.loop(0, nch // 4)
        def _(h):
            for bb in range(4):
                c = h * 4 + bb
                b2 = bb % 2

                @pl.when(c + 1 < nch)
                def _():
                    idx_wait(c + 1, (bb + 1) % 4)
                    g_start((bb + 1) % 4, (bb + 1) % 2)
                    a_start(c + 1, (bb + 1) % 2)

                @pl.when(c + 2 < nch)
                def _():
                    idx_start(c + 2, (bb + 2) % 4)

                g_wait(bb, b2)
                a_wait(c, b2)

                @pl.when(c >= 2)
                def _():
                    pltpu.make_async_copy(o_v[b2], o_slice(0), osem).wait()

                rv = rows_v[b2]
                av = a_v[b2]
                ov = o_v[b2]
                for n in range(ch_nodes):
                    for cc in range(ncols):
                        sl = pl.ds(cc * 16, 16)
                        acc = rv[n * K, sl]
                        for j in range(1, K):
                            acc = acc + rv[n * K + j, sl]
                        ov[n, sl] = acc * jnp.float32(1.0 / K) + av[n, sl]
                pltpu.async_copy(ov, o_slice(c), osem)

        pltpu.make_async_copy(o_v[0], o_slice(0), osem).wait()
        pltpu.make_async_copy(o_v[1], o_slice(0), osem).wait()

    return meanadd


_make_gather = functools.lru_cache(maxsize=None)(_make_gather)
_make_meanadd = functools.lru_cache(maxsize=None)(_make_meanadd)


def kernel(X, W1, b1, W2, b2):
    Xbf = X.astype(jnp.bfloat16)
    sq1 = jnp.sum(X * X, axis=-1)
    idx1 = _topk(Xbf, X.T.astype(jnp.bfloat16), sq1)
    idx1_km = idx1.T.reshape(-1)
    XJ = _make_gather(N * K, 128, 256, N)(X, idx1_km).reshape(K, N, 128)
    F = _edgeconv(X, XJ, W1, b1)
    sq2 = jnp.sum(F * F, axis=-1)
    idx2 = _topk(F.astype(jnp.bfloat16), F.T.astype(jnp.bfloat16), sq2)
    S = _make_meanadd(8)(F, F, idx2.reshape(-1))
    return _final(S, W2, b2)
